# baseline (speedup 1.0000x reference)
import functools

import jax
import jax.numpy as jnp
from jax import lax
from jax.experimental import pallas as pl
from jax.experimental.pallas import tpu as pltpu
from jax.experimental.pallas import tpu_sc as plsc

N_TOTAL = 50000
EMB = 64
LAYERS = 3
E_EDGES = 800000
B_PAIRS = 4096

NC = 2
NS = 16
L = 16

R_HALF = 25088
N_PAD = 2 * R_HALF
ROWS_PER_TILE = R_HALF // NS
K_EDGE = 128
NB = E_EDGES // K_EDGE
SB_E = 128
CHUNK_E = 1024
E_PAD = E_EDGES + 2 * CHUNK_E
PAIRS_PER_TILE = B_PAIRS // (NC * NS)

_MESH = plsc.VectorSubcoreMesh(core_axis_name="c", subcore_axis_name="s")


@functools.partial(
    pl.kernel,
    mesh=_MESH,
    out_type=jax.ShapeDtypeStruct((N_PAD, EMB), jnp.float32),
    scratch_types=[
        pltpu.VMEM((L,), jnp.int32),
        pltpu.VMEM((CHUNK_E,), jnp.int32),
        pltpu.VMEM((CHUNK_E,), jnp.int32),
        pltpu.VMEM((CHUNK_E,), jnp.float32),
        pltpu.VMEM((1, K_EDGE), jnp.int32),
        pltpu.VMEM((1, K_EDGE), jnp.int32),
        pltpu.VMEM((SB_E, EMB), jnp.float32),
        pltpu.VMEM((SB_E, EMB), jnp.float32),
        pltpu.SemaphoreType.DMA,
        pltpu.SemaphoreType.DMA,
        pltpu.SemaphoreType.DMA,
        pltpu.SemaphoreType.DMA,
        pltpu.VMEM_SHARED((R_HALF + 8, EMB), jnp.float32),
    ],
    compiler_params=pltpu.CompilerParams(use_tc_tiling_on_sc=False, needs_layout_passes=False),
)
def _spmm(params_hbm, col_hbm, row_hbm, val_hbm, ego_hbm, msg_hbm,
          params_v, col_c, row_c, val_c, idx2_0, idx2_1, rows_0, rows_1,
          gsem0, gsem1, ssem0, ssem1, acc_sh):
    cid = lax.axis_index("c")
    sid = lax.axis_index("s")
    base_row = cid * R_HALF
    rows_b = (rows_0, rows_1)
    idx2_b = (idx2_0, idx2_1)
    gsem = (gsem0, gsem1)
    ssem = (ssem0, ssem1)
    lane = lax.iota(jnp.int32, L)

    z = jnp.zeros((L,), jnp.float32)
    for j in range(K_EDGE):
        for c in range(EMB // L):
            rows_0[j, pl.ds(c * L, L)] = z
    my0 = sid * ROWS_PER_TILE
    nfull = ROWS_PER_TILE // K_EDGE
    rem = ROWS_PER_TILE - nfull * K_EDGE
    for t in range(nfull):
        pltpu.sync_copy(rows_0, acc_sh.at[pl.ds(my0 + t * K_EDGE, K_EDGE)])
    pltpu.sync_copy(rows_0.at[pl.ds(0, rem)], acc_sh.at[pl.ds(my0 + nfull * K_EDGE, rem)])
    pltpu.sync_copy(rows_0.at[pl.ds(0, 8)], acc_sh.at[pl.ds(R_HALF, 8)])

    pltpu.sync_copy(params_hbm, params_v)
    e_mid = params_v[pl.ds(0, L)][0]
    b_lo = e_mid // K_EDGE
    b_hi = (e_mid + K_EDGE - 1) // K_EDGE
    b_start = jnp.where(cid == 0, 0, b_lo)
    b_end = jnp.where(cid == 0, b_hi, NB)
    nb = b_end - b_start
    cpt = (nb + NS - 1) // NS
    my_first_blk = b_start + sid * cpt
    my_e0 = my_first_blk * K_EDGE
    my_e1 = jnp.minimum(my_first_blk + cpt, b_end) * K_EDGE
    ns_sb = cpt
    nc = jnp.maximum(1, (ns_sb + 7) // 8)

    def stage_adj(c):
        off = my_e0 + c * CHUNK_E
        pltpu.sync_copy(col_hbm.at[pl.ds(off, CHUNK_E)], col_c)
        pltpu.sync_copy(row_hbm.at[pl.ds(off, CHUNK_E)], row_c)
        pltpu.sync_copy(val_hbm.at[pl.ds(off, CHUNK_E)], val_c)

    def fire_gathers(j, p):
        for q in range(SB_E // K_EDGE):
            pltpu.async_copy(
                ego_hbm.at[col_c.at[pl.ds(j * SB_E + q * K_EDGE, K_EDGE)]],
                rows_b[p].at[pl.ds(q * K_EDGE, K_EDGE)], gsem[p])

    def drain_gathers(p):
        for q in range(SB_E // K_EDGE):
            pltpu.make_async_copy(
                ego_hbm.at[col_c.at[pl.ds(q * K_EDGE, K_EDGE)]],
                rows_b[p].at[pl.ds(q * K_EDGE, K_EDGE)], gsem[p]).wait()

    def fire_scatters(p):
        for q in range(SB_E // K_EDGE):
            pltpu.async_copy(rows_b[p].at[pl.ds(q * K_EDGE, K_EDGE)],
                             acc_sh.at[idx2_b[p].at[q]], ssem[p], add=True)

    def drain_scatters(p):
        for q in range(SB_E // K_EDGE):
            pltpu.make_async_copy(rows_b[p].at[pl.ds(q * K_EDGE, K_EDGE)],
                                  acc_sh.at[idx2_b[p].at[q]], ssem[p]).wait()

    def process_sb(c, j, p):
        def g_body(g, carry):
            o16 = j * SB_E + g * L
            val16 = val_c[pl.ds(o16, L)]
            row16 = row_c[pl.ds(o16, L)]
            lr = row16 - base_row
            eid = (my_e0 + c * CHUNK_E + o16) + lane
            ok = (lr >= 0) & (lr < R_HALF) & (eid < my_e1)
            idxm = jnp.where(ok, lr, R_HALF)
            q = g // 8
            r = (g - q * 8) * L + lane
            plsc.store_scatter(idx2_b[p], [jnp.full((L,), q, jnp.int32), r], idxm)
            el = g * L + lane
            for ccc in range(EMB):
                colv = jnp.full((L,), ccc, jnp.int32)
                x = plsc.load_gather(rows_b[p], [el, colv])
                plsc.store_scatter(rows_b[p], [el, colv], x * val16)
            return carry
        lax.fori_loop(0, SB_E // L, g_body, 0)

    stage_adj(0)
    fire_gathers(0, 0)
    plsc.subcore_barrier()

    def chunk_body(c, carry):
        for j in range(CHUNK_E // SB_E):
            p = j & 1
            drain_gathers(p)
            process_sb(c, j, p)
            if j < 7:
                if j == 0:
                    @pl.when(c > 0)
                    def _():
                        drain_scatters(1)
                else:
                    drain_scatters(1 - p)
                fire_gathers(j + 1, 1 - p)
            else:
                @pl.when(c + 1 < nc)
                def _():
                    stage_adj(c + 1)
                    drain_scatters(0)
                    fire_gathers(0, 0)
            fire_scatters(p)
        return carry

    lax.fori_loop(0, nc, chunk_body, 0)
    drain_scatters(0)
    drain_scatters(1)
    plsc.subcore_barrier()

    pltpu.sync_copy(acc_sh.at[pl.ds(my0, ROWS_PER_TILE)],
                    msg_hbm.at[pl.ds(base_row + my0, ROWS_PER_TILE)])


def _dense_body(msg_ref, ego_ref, gw_ref, gb_ref, bw_ref, bb_ref,
                ego_out_ref, norm_ref):
    msg = msg_ref[...]
    ego = ego_ref[...]
    aggr = lax.dot_general(msg, gw_ref[...], (((1,), (1,)), ((), ())),
                           preferred_element_type=jnp.float32) + gb_ref[...]
    bi = lax.dot_general(ego * msg, bw_ref[...], (((1,), (1,)), ((), ())),
                         preferred_element_type=jnp.float32) + bb_ref[...]
    h = aggr + bi
    h = jnp.where(h >= 0, h, 0.2 * h)
    ego_out_ref[...] = h
    n = jnp.sqrt(jnp.sum(h * h, axis=1, keepdims=True))
    norm_ref[...] = h / jnp.maximum(n, 1e-12)


TC_BLK = 512

_dense = pl.pallas_call(
    _dense_body,
    grid=(N_PAD // TC_BLK,),
    in_specs=[
        pl.BlockSpec((TC_BLK, EMB), lambda i: (i, 0)),
        pl.BlockSpec((TC_BLK, EMB), lambda i: (i, 0)),
        pl.BlockSpec((EMB, EMB), lambda i: (0, 0)),
        pl.BlockSpec((1, EMB), lambda i: (0, 0)),
        pl.BlockSpec((EMB, EMB), lambda i: (0, 0)),
        pl.BlockSpec((1, EMB), lambda i: (0, 0)),
    ],
    out_specs=[
        pl.BlockSpec((TC_BLK, EMB), lambda i: (i, 0)),
        pl.BlockSpec((TC_BLK, EMB), lambda i: (i, 0)),
    ],
    out_shape=[
        jax.ShapeDtypeStruct((N_PAD, EMB), jnp.float32),
        jax.ShapeDtypeStruct((N_PAD, EMB), jnp.float32),
    ],
)


@functools.partial(
    pl.kernel,
    mesh=_MESH,
    out_type=jax.ShapeDtypeStruct((B_PAIRS,), jnp.float32),
    scratch_types=[
        pltpu.VMEM((PAIRS_PER_TILE,), jnp.int32),
        pltpu.VMEM((PAIRS_PER_TILE,), jnp.int32),
        pltpu.VMEM((PAIRS_PER_TILE,), jnp.float32),
        pltpu.VMEM((PAIRS_PER_TILE, EMB), jnp.float32),
        pltpu.VMEM((PAIRS_PER_TILE, EMB), jnp.float32),
        pltpu.VMEM((PAIRS_PER_TILE, EMB), jnp.float32),
        pltpu.VMEM((PAIRS_PER_TILE, EMB), jnp.float32),
        pltpu.VMEM((PAIRS_PER_TILE, EMB), jnp.float32),
        pltpu.VMEM((PAIRS_PER_TILE, EMB), jnp.float32),
        pltpu.VMEM((PAIRS_PER_TILE, EMB), jnp.float32),
        pltpu.VMEM((PAIRS_PER_TILE, EMB), jnp.float32),
        pltpu.SemaphoreType.DMA,
    ],
    compiler_params=pltpu.CompilerParams(use_tc_tiling_on_sc=False, needs_layout_passes=False),
)
def _pair_dot(si_hbm, di_hbm, t0, t1, t2, t3, out_hbm,
              is_v, id_v, out_v, rs0, rs1, rs2, rs3, rd0, rd1, rd2, rd3, sem):
    cid = lax.axis_index("c")
    sid = lax.axis_index("s")
    wid = sid * NC + cid
    base = wid * PAIRS_PER_TILE
    pltpu.sync_copy(si_hbm.at[pl.ds(base, PAIRS_PER_TILE)], is_v)
    pltpu.sync_copy(di_hbm.at[pl.ds(base, PAIRS_PER_TILE)], id_v)
    for tbl, rs, rd in ((t0, rs0, rd0), (t1, rs1, rd1), (t2, rs2, rd2), (t3, rs3, rd3)):
        pltpu.async_copy(tbl.at[is_v], rs, sem).wait()
        pltpu.async_copy(tbl.at[id_v], rd, sem).wait()

    lane = lax.iota(jnp.int32, L)
    for g in range(PAIRS_PER_TILE // L):
        pair = lane + g * L
        acc = jnp.zeros((L,), jnp.float32)
        for rs, rd in ((rs0, rd0), (rs1, rd1), (rs2, rd2), (rs3, rd3)):
            for c in range(EMB):
                ccol = jnp.full((L,), c, jnp.int32)
                a = plsc.load_gather(rs, [pair, ccol])
                b = plsc.load_gather(rd, [pair, ccol])
                acc = acc + a * b
        out_v[pl.ds(g * L, L)] = acc
    pltpu.sync_copy(out_v, out_hbm.at[pl.ds(base, PAIRS_PER_TILE)])


def kernel(edge_label_index, adj_row, adj_col, adj_value, emb,
           gc_w0, gc_b0, bi_w0, bi_b0,
           gc_w1, gc_b1, bi_w1, bi_b1,
           gc_w2, gc_b2, bi_w2, bi_b2):
    e_mid = jnp.searchsorted(adj_row, jnp.int32(R_HALF), side="left").astype(jnp.int32)
    params = jnp.zeros((L,), jnp.int32).at[0].set(e_mid)
    npad = E_PAD - E_EDGES
    col_p = jnp.concatenate([adj_col, jnp.zeros((npad,), jnp.int32)])
    row_p = jnp.concatenate([adj_row, jnp.full((npad,), N_PAD, jnp.int32)])
    val_p = jnp.concatenate([adj_value, jnp.zeros((npad,), jnp.float32)])
    ego = jnp.zeros((N_PAD, EMB), jnp.float32).at[:N_TOTAL].set(emb)
    gc = [(gc_w0, gc_b0), (gc_w1, gc_b1), (gc_w2, gc_b2)]
    bi = [(bi_w0, bi_b0), (bi_w1, bi_b1), (bi_w2, bi_b2)]
    tables = [ego]
    for i in range(LAYERS):
        msg = _spmm(params, col_p, row_p, val_p, ego)
        ego, norm = _dense(msg, ego, gc[i][0], gc[i][1].reshape(1, EMB),
                           bi[i][0], bi[i][1].reshape(1, EMB))
        tables.append(norm)
    return _pair_dot(edge_label_index[0], edge_label_index[1],
                     tables[0], tables[1], tables[2], tables[3])
